# baseline (speedup 1.0000x reference)
import jax
import jax.numpy as jnp
from jax.experimental import pallas as pl
from jax.experimental.pallas import tpu as pltpu

K = 512
D = 32
BETA = 0.25
BN = 2048


def _vq_kernel(z_ref, cb_ref, ut_ref, out_ref, loss_ref):
    i = pl.program_id(0)
    z = z_ref[...]
    cb = cb_ref[...]
    z2 = jnp.sum(z * z, axis=1, keepdims=True)
    e2 = jnp.sum(cb * cb, axis=1)[None, :]
    cross = jax.lax.dot_general(
        z, cb, (((1,), (1,)), ((), ())),
        preferred_element_type=jnp.float32,
        precision=jax.lax.Precision.DEFAULT)
    dist = z2 - 2.0 * cross + e2
    minv = jnp.min(dist, axis=1, keepdims=True)
    m = (dist <= minv).astype(jnp.float32)
    pc = jax.lax.dot_general(
        m, ut_ref[...], (((1,), (0,)), ((), ())),
        preferred_element_type=jnp.float32,
        precision=jax.lax.Precision.DEFAULT)
    onehot = jnp.where(pc == 0.0, m, 0.0)
    q = jax.lax.dot_general(
        onehot, cb, (((1,), (0,)), ((), ())),
        preferred_element_type=jnp.float32,
        precision=jax.lax.Precision.HIGHEST)
    out_ref[...] = z + (q - z)
    psum = jnp.sum((q - z) ** 2)

    @pl.when(i == 0)
    def _init():
        loss_ref[0, 0] = 0.0

    loss_ref[0, 0] += psum


def kernel(z, codebook):
    n = z.shape[0] * z.shape[1]
    flat = z.reshape(n, D)
    grid = n // BN
    col = jax.lax.broadcasted_iota(jnp.int32, (K, K), 1)
    row = jax.lax.broadcasted_iota(jnp.int32, (K, K), 0)
    upper_tri = (row < col).astype(jnp.float32)
    out, loss_sum = pl.pallas_call(
        _vq_kernel,
        grid=(grid,),
        in_specs=[
            pl.BlockSpec((BN, D), lambda i: (i, 0)),
            pl.BlockSpec((K, D), lambda i: (0, 0)),
            pl.BlockSpec((K, K), lambda i: (0, 0)),
        ],
        out_specs=[
            pl.BlockSpec((BN, D), lambda i: (i, 0)),
            pl.BlockSpec(memory_space=pltpu.SMEM),
        ],
        out_shape=[
            jax.ShapeDtypeStruct((n, D), jnp.float32),
            jax.ShapeDtypeStruct((1, 1), jnp.float32),
        ],
    )(flat, codebook, upper_tri)
    mse = loss_sum[0, 0] / jnp.float32(n * D)
    loss = (1.0 + BETA) * mse
    return out.reshape(z.shape), loss


# speedup vs baseline: 1.0831x; 1.0831x over previous
import jax
import jax.numpy as jnp
from jax.experimental import pallas as pl
from jax.experimental.pallas import tpu as pltpu

K = 512
D = 32
BETA = 0.25
BN = 2048


def _vq_kernel(z_ref, cb_ref, ones_ref, out_ref, loss_ref):
    i = pl.program_id(0)
    z = z_ref[...]
    cb = cb_ref[...]
    z2 = jnp.sum(z * z, axis=1, keepdims=True)
    e2 = jnp.sum(cb * cb, axis=1)[None, :]
    cross = jax.lax.dot_general(
        z, cb, (((1,), (1,)), ((), ())),
        preferred_element_type=jnp.float32,
        precision=jax.lax.Precision.DEFAULT)
    dist = z2 - 2.0 * cross + e2
    minv = jnp.min(dist, axis=1, keepdims=True)
    iota = jax.lax.broadcasted_iota(jnp.int32, dist.shape, 1)
    idx = jnp.min(jnp.where(dist <= minv, iota, K), axis=1)
    idx_bf = (idx[:, None] - 256).astype(jnp.bfloat16)
    idx_bcast = jax.lax.dot_general(
        idx_bf, ones_ref[...], (((1,), (0,)), ((), ())),
        preferred_element_type=jnp.float32)
    onehot = (iota.astype(jnp.float32) - 256.0 == idx_bcast).astype(jnp.float32)
    q = jax.lax.dot_general(
        onehot, cb, (((1,), (0,)), ((), ())),
        preferred_element_type=jnp.float32,
        precision=jax.lax.Precision.HIGHEST)
    out_ref[...] = z + (q - z)
    psum = jnp.sum((q - z) ** 2)

    @pl.when(i == 0)
    def _init():
        loss_ref[0, 0] = 0.0

    loss_ref[0, 0] += psum


def kernel(z, codebook):
    n = z.shape[0] * z.shape[1]
    flat = z.reshape(n, D)
    grid = n // BN
    ones_row = jnp.ones((1, K), dtype=jnp.bfloat16)
    out, loss_sum = pl.pallas_call(
        _vq_kernel,
        grid=(grid,),
        in_specs=[
            pl.BlockSpec((BN, D), lambda i: (i, 0)),
            pl.BlockSpec((K, D), lambda i: (0, 0)),
            pl.BlockSpec((1, K), lambda i: (0, 0)),
        ],
        out_specs=[
            pl.BlockSpec((BN, D), lambda i: (i, 0)),
            pl.BlockSpec(memory_space=pltpu.SMEM),
        ],
        out_shape=[
            jax.ShapeDtypeStruct((n, D), jnp.float32),
            jax.ShapeDtypeStruct((1, 1), jnp.float32),
        ],
    )(flat, codebook, ones_row)
    mse = loss_sum[0, 0] / jnp.float32(n * D)
    loss = (1.0 + BETA) * mse
    return out.reshape(z.shape), loss


# speedup vs baseline: 1.6042x; 1.4810x over previous
import jax
import jax.numpy as jnp
from jax.experimental import pallas as pl
from jax.experimental.pallas import tpu as pltpu

K = 512
D = 32
BETA = 0.25
BN = 2048


def _vq_kernel(z_ref, cb_ref, ones_ref, iota_ref, cbb_ref, out_ref, loss_ref):
    i = pl.program_id(0)
    z = z_ref[...]
    cb = cb_ref[...]
    z2 = jnp.sum(z * z, axis=1, keepdims=True)
    e2 = jnp.sum(cb * cb, axis=1)[None, :]
    cross = jax.lax.dot_general(
        z, cb, (((1,), (1,)), ((), ())),
        preferred_element_type=jnp.float32,
        precision=jax.lax.Precision.DEFAULT)
    dist = z2 - 2.0 * cross + e2
    minv = jnp.min(dist, axis=1, keepdims=True)
    iota = jax.lax.broadcasted_iota(jnp.int32, dist.shape, 1)
    idx = jnp.min(jnp.where(dist <= minv, iota, K), axis=1)
    idx_bf = (idx[:, None] - 256).astype(jnp.bfloat16)
    idx_bcast = jax.lax.dot_general(
        idx_bf, ones_ref[...], (((1,), (0,)), ((), ())),
        preferred_element_type=jnp.float32)
    onehot = jnp.where(iota_ref[...] == idx_bcast,
                       1.0, 0.0).astype(jnp.bfloat16)
    q = jax.lax.dot_general(
        onehot, cbb_ref[...], (((1,), (0,)), ((), ())),
        preferred_element_type=jnp.float32)
    out_ref[...] = z + (q - z)
    psum = jnp.sum((q - z) ** 2)

    @pl.when(i == 0)
    def _init():
        loss_ref[0, 0] = 0.0

    loss_ref[0, 0] += psum


def kernel(z, codebook):
    n = z.shape[0] * z.shape[1]
    flat = z.reshape(n, D)
    grid = n // BN
    ones_row = jnp.ones((1, K), dtype=jnp.bfloat16)
    iota_row = (jnp.arange(K, dtype=jnp.float32) - 256.0).reshape(1, K)
    cb_bf = codebook.astype(jnp.bfloat16)
    out, loss_sum = pl.pallas_call(
        _vq_kernel,
        grid=(grid,),
        in_specs=[
            pl.BlockSpec((BN, D), lambda i: (i, 0)),
            pl.BlockSpec((K, D), lambda i: (0, 0)),
            pl.BlockSpec((1, K), lambda i: (0, 0)),
            pl.BlockSpec((1, K), lambda i: (0, 0)),
            pl.BlockSpec((K, D), lambda i: (0, 0)),
        ],
        out_specs=[
            pl.BlockSpec((BN, D), lambda i: (i, 0)),
            pl.BlockSpec(memory_space=pltpu.SMEM),
        ],
        out_shape=[
            jax.ShapeDtypeStruct((n, D), jnp.float32),
            jax.ShapeDtypeStruct((1, 1), jnp.float32),
        ],
    )(flat, codebook, ones_row, iota_row, cb_bf)
    mse = loss_sum[0, 0] / jnp.float32(n * D)
    loss = (1.0 + BETA) * mse
    return out.reshape(z.shape), loss


# speedup vs baseline: 1.8706x; 1.1661x over previous
import jax
import jax.numpy as jnp
from jax.experimental import pallas as pl
from jax.experimental.pallas import tpu as pltpu

K = 512
D = 32
BETA = 0.25
BN = 4096


def _vq_kernel(z_ref, cb_ref, ones_ref, iota_ref, cbb_ref, out_ref, loss_ref):
    i = pl.program_id(0)
    z = z_ref[...]
    cb = cb_ref[...]
    z2 = jnp.sum(z * z, axis=1, keepdims=True)
    e2 = jnp.sum(cb * cb, axis=1)[None, :]
    cross = jax.lax.dot_general(
        z, cb, (((1,), (1,)), ((), ())),
        preferred_element_type=jnp.float32,
        precision=jax.lax.Precision.DEFAULT)
    dist = z2 - 2.0 * cross + e2
    minv = jnp.min(dist, axis=1, keepdims=True)
    idxs = jnp.min(jnp.where(dist <= minv, iota_ref[...], 256.0),
                   axis=1, keepdims=True)
    idx_bcast = jax.lax.dot_general(
        idxs.astype(jnp.bfloat16), ones_ref[...], (((1,), (0,)), ((), ())),
        preferred_element_type=jnp.float32)
    onehot = jnp.where(iota_ref[...] == idx_bcast,
                       1.0, 0.0).astype(jnp.bfloat16)
    q = jax.lax.dot_general(
        onehot, cbb_ref[...], (((1,), (0,)), ((), ())),
        preferred_element_type=jnp.float32)
    out_ref[...] = q
    psum = jnp.sum(minv)

    @pl.when(i == 0)
    def _init():
        loss_ref[0, 0] = 0.0

    loss_ref[0, 0] += psum


def kernel(z, codebook):
    n = z.shape[0] * z.shape[1]
    flat = z.reshape(n, D)
    grid = n // BN
    ones_row = jnp.ones((1, K), dtype=jnp.bfloat16)
    iota_row = (jnp.arange(K, dtype=jnp.float32) - 256.0).reshape(1, K)
    cb_bf = codebook.astype(jnp.bfloat16)
    out, loss_sum = pl.pallas_call(
        _vq_kernel,
        grid=(grid,),
        in_specs=[
            pl.BlockSpec((BN, D), lambda i: (i, 0)),
            pl.BlockSpec((K, D), lambda i: (0, 0)),
            pl.BlockSpec((1, K), lambda i: (0, 0)),
            pl.BlockSpec((1, K), lambda i: (0, 0)),
            pl.BlockSpec((K, D), lambda i: (0, 0)),
        ],
        out_specs=[
            pl.BlockSpec((BN, D), lambda i: (i, 0)),
            pl.BlockSpec(memory_space=pltpu.SMEM),
        ],
        out_shape=[
            jax.ShapeDtypeStruct((n, D), jnp.float32),
            jax.ShapeDtypeStruct((1, 1), jnp.float32),
        ],
    )(flat, codebook, ones_row, iota_row, cb_bf)
    mse = loss_sum[0, 0] / jnp.float32(n * D)
    loss = (1.0 + BETA) * mse
    return out.reshape(z.shape), loss


# speedup vs baseline: 1.8809x; 1.0055x over previous
import jax
import jax.numpy as jnp
from jax.experimental import pallas as pl
from jax.experimental.pallas import tpu as pltpu

K = 512
D = 32
BETA = 0.25
BN = 8192


def _vq_kernel(z_ref, cb_ref, ones_ref, iota_ref, cbb_ref, out_ref, loss_ref):
    i = pl.program_id(0)
    z = z_ref[...]
    cb = cb_ref[...]
    z2 = jnp.sum(z * z, axis=1, keepdims=True)
    e2 = jnp.sum(cb * cb, axis=1)[None, :]
    cross = jax.lax.dot_general(
        z, cb, (((1,), (1,)), ((), ())),
        preferred_element_type=jnp.float32,
        precision=jax.lax.Precision.DEFAULT)
    dist = z2 - 2.0 * cross + e2
    minv = jnp.min(dist, axis=1, keepdims=True)
    idxs = jnp.min(jnp.where(dist <= minv, iota_ref[...], 256.0),
                   axis=1, keepdims=True)
    idx_bcast = jax.lax.dot_general(
        idxs.astype(jnp.bfloat16), ones_ref[...], (((1,), (0,)), ((), ())),
        preferred_element_type=jnp.float32)
    onehot = jnp.where(iota_ref[...] == idx_bcast,
                       1.0, 0.0).astype(jnp.bfloat16)
    q = jax.lax.dot_general(
        onehot, cbb_ref[...], (((1,), (0,)), ((), ())),
        preferred_element_type=jnp.float32)
    out_ref[...] = q
    psum = jnp.sum(minv)

    @pl.when(i == 0)
    def _init():
        loss_ref[0, 0] = 0.0

    loss_ref[0, 0] += psum


def kernel(z, codebook):
    n = z.shape[0] * z.shape[1]
    flat = z.reshape(n, D)
    grid = n // BN
    ones_row = jnp.ones((1, K), dtype=jnp.bfloat16)
    iota_row = (jnp.arange(K, dtype=jnp.float32) - 256.0).reshape(1, K)
    cb_bf = codebook.astype(jnp.bfloat16)
    out, loss_sum = pl.pallas_call(
        _vq_kernel,
        grid=(grid,),
        in_specs=[
            pl.BlockSpec((BN, D), lambda i: (i, 0)),
            pl.BlockSpec((K, D), lambda i: (0, 0)),
            pl.BlockSpec((1, K), lambda i: (0, 0)),
            pl.BlockSpec((1, K), lambda i: (0, 0)),
            pl.BlockSpec((K, D), lambda i: (0, 0)),
        ],
        out_specs=[
            pl.BlockSpec((BN, D), lambda i: (i, 0)),
            pl.BlockSpec(memory_space=pltpu.SMEM),
        ],
        out_shape=[
            jax.ShapeDtypeStruct((n, D), jnp.float32),
            jax.ShapeDtypeStruct((1, 1), jnp.float32),
        ],
    )(flat, codebook, ones_row, iota_row, cb_bf)
    mse = loss_sum[0, 0] / jnp.float32(n * D)
    loss = (1.0 + BETA) * mse
    return out.reshape(z.shape), loss


# speedup vs baseline: 2.1776x; 1.1578x over previous
import jax
import jax.numpy as jnp
from jax.experimental import pallas as pl
from jax.experimental.pallas import tpu as pltpu

K = 512
D = 32
BETA = 0.25
BG = 8


def _vq_kernel(z_ref, cb_ref, out_ref, loss_ref):
    i = pl.program_id(0)
    ng = pl.num_programs(0)
    bn = BG * z_ref.shape[1]
    z = z_ref[...].reshape(bn, D)
    cb = cb_ref[...]
    z2 = jnp.sum(z * z, axis=1, keepdims=True)
    e2 = jnp.sum(cb * cb, axis=1)[None, :]
    cross = jax.lax.dot_general(
        z, cb, (((1,), (1,)), ((), ())),
        preferred_element_type=jnp.float32,
        precision=jax.lax.Precision.DEFAULT)
    dist = z2 - 2.0 * cross + e2
    minv = jnp.min(dist, axis=1, keepdims=True)
    iota_row = jax.lax.broadcasted_iota(
        jnp.int32, (1, K), 1).astype(jnp.float32) - 256.0
    idxs = jnp.min(jnp.where(dist <= minv, iota_row, 256.0),
                   axis=1, keepdims=True)
    idx_bcast = jax.lax.dot_general(
        idxs.astype(jnp.bfloat16), jnp.ones((1, K), jnp.bfloat16),
        (((1,), (0,)), ((), ())),
        preferred_element_type=jnp.float32)
    onehot = jnp.where(iota_row == idx_bcast,
                       1.0, 0.0).astype(jnp.bfloat16)
    q = jax.lax.dot_general(
        onehot, cb.astype(jnp.bfloat16), (((1,), (0,)), ((), ())),
        preferred_element_type=jnp.float32)
    out_ref[...] = q.reshape(out_ref.shape)
    psum = jnp.sum(minv)

    @pl.when(i == 0)
    def _init():
        loss_ref[0, 0] = 0.0

    loss_ref[0, 0] += psum

    @pl.when(i == ng - 1)
    def _finish():
        scale = (1.0 + BETA) / jnp.float32(ng * bn * D)
        loss_ref[0, 0] = loss_ref[0, 0] * scale


def kernel(z, codebook):
    g, s, _ = z.shape
    out, loss = pl.pallas_call(
        _vq_kernel,
        grid=(g // BG,),
        in_specs=[
            pl.BlockSpec((BG, s, D), lambda i: (i, 0, 0)),
            pl.BlockSpec((K, D), lambda i: (0, 0)),
        ],
        out_specs=[
            pl.BlockSpec((BG, s, D), lambda i: (i, 0, 0)),
            pl.BlockSpec(memory_space=pltpu.SMEM),
        ],
        out_shape=[
            jax.ShapeDtypeStruct(z.shape, jnp.float32),
            jax.ShapeDtypeStruct((1, 1), jnp.float32),
        ],
    )(z, codebook)
    return out, loss[0, 0]
